# baseline (speedup 1.0000x reference)
import jax
import jax.numpy as jnp
from jax.experimental import pallas as pl
from jax.experimental.pallas import tpu as pltpu
from jax.experimental.pallas import tpu_sc as plsc

_B = 16
_N = 4096
_ENC_IN = 128
_D = 128
_HALF = _D // 2
_PATCH_LEN = 128
_NP = 32
_Q = 64
_NB = 4096
_NC = _N // _NB
_SCL = 16
_SCN = 2048

_TRANS_LHS = (((0,), (0,)), ((), ()))


_CH = 512


def _fused_body(x4t_ref, qm_row_ref, r_row_ref, t_row_ref,
                m_row_ref, vw_ref, tt_ref, vtab_ref, ptab_ref,
                obs_ref, ph_ref, vh_ref, pinc_ref):
    f32 = jnp.float32
    bf16 = jnp.bfloat16
    sub = jax.lax.broadcasted_iota(jnp.int32, (8, _CH), 0)

    for c in range(_NB // _CH):
        sl = pl.ds(c * _CH, _CH)

        val = jax.lax.dot_general(x4t_ref[0, :, sl], vw_ref[...], _TRANS_LHS,
                                  preferred_element_type=f32)
        obs_ref[0, sl, 0:_HALF] = val

        d8q = jnp.broadcast_to(qm_row_ref[0, :, sl], (8, _CH)) - sub
        d8r = jnp.broadcast_to(r_row_ref[0, :, sl], (8, _CH)) - sub
        parts = [(d8q == 8 * k).astype(f32) for k in range(_Q // 8)]
        parts += [(d8r == 8 * k).astype(f32) for k in range(_Q // 8)]
        oht = jnp.concatenate(parts, axis=0).astype(bf16)
        og = jax.lax.dot_general(oht, tt_ref[...], _TRANS_LHS,
                                 preferred_element_type=f32).astype(bf16)
        p = og[:, 0:_D] * og[:, _D:2 * _D]
        time_feat = p[:, 0:_HALF] + p[:, _HALF:_D]
        obs_ref[0, sl, _HALF:_D] = time_feat.astype(f32)

        m8 = jnp.broadcast_to(m_row_ref[0, :, sl].astype(f32), (8, _CH))
        d8p = jnp.broadcast_to(t_row_ref[0, :, sl] // _PATCH_LEN,
                               (8, _CH)) - sub
        for k in range(_NP // 8):
            pinc_ref[0, 8 * k:8 * (k + 1), sl] = jnp.where(
                d8p == 8 * k, m8, 0.0)

    vh_ref[0] = vtab_ref[...]
    ph_ref[0] = ptab_ref[...]


def kernel(x_flattened, time_indices_flattened, variable_indices_flattened,
           observation_mask_flattened, W_val, b_val, W_time, b_time,
           variable_hyperedge_embedding, patch_hyperedge_embedding):
    f32 = jnp.float32
    bf16 = jnp.bfloat16
    t_i = time_indices_flattened
    m_i = observation_mask_flattened

    qm_row = jnp.where(m_i != 0, t_i >> 6, -1).reshape(_B, 1, _N)
    r_row = (t_i & (_Q - 1)).reshape(_B, 1, _N)
    xm = x_flattened * m_i.astype(f32)
    xh = xm.astype(bf16)
    xl = (xm - xh.astype(f32)).astype(bf16)
    x4t = jnp.stack([xh, xh, xl, m_i.astype(bf16)], axis=1)

    t_row = t_i.reshape(_B, 1, _N)
    v_row = variable_indices_flattened.reshape(_B, 1, _N)
    m_row = m_i.reshape(_B, 1, _N)

    wh = W_val.astype(bf16)
    wl = (W_val - wh.astype(f32)).astype(bf16)
    vw4 = jnp.concatenate([wh, wl, wh, b_val.astype(bf16)[None]], axis=0)

    w_t = W_time[0]
    steps = jnp.arange(_Q, dtype=f32)[:, None]
    a_tab = (_Q * steps) * w_t[None, :]
    b_tab = steps * w_t[None, :] + b_time[None, :]
    qt = jnp.concatenate([jnp.sin(a_tab), jnp.cos(a_tab)], axis=1)
    rt = jnp.concatenate([jnp.cos(b_tab), jnp.sin(b_tab)], axis=1)
    zz = jnp.zeros((_Q, _D), f32)
    t_big = jnp.block([[qt, zz], [zz, rt]]).astype(bf16)

    row_spec = pl.BlockSpec((1, 1, _NB), lambda b, c: (b, 0, c))
    small = lambda shape: pl.BlockSpec(shape, lambda b, c: (0,) * len(shape))

    out_types = (
        jax.ShapeDtypeStruct((_B, _N, _D), f32),
        jax.ShapeDtypeStruct((_B, _NP, _D), f32),
        jax.ShapeDtypeStruct((_B, _ENC_IN, _D), f32),
        jax.ShapeDtypeStruct((_B, _NP, _N), f32),
    )
    out_specs = (
        pl.BlockSpec((1, _NB, _D), lambda b, c: (b, c, 0)),
        pl.BlockSpec((1, _NP, _D), lambda b, c: (b, 0, 0)),
        pl.BlockSpec((1, _ENC_IN, _D), lambda b, c: (b, 0, 0)),
        pl.BlockSpec((1, _NP, _NB), lambda b, c: (b, 0, c)),
    )
    in_specs = [
        pl.BlockSpec((1, 4, _NB), lambda b, c: (b, 0, c)),
        row_spec, row_spec, row_spec, row_spec,
        small((4, _HALF)), small((_D, 2 * _D)),
        small((_ENC_IN, _D)), small((_NP, _D)),
    ]

    obs, ph, vh, pinc = pl.pallas_call(
        _fused_body,
        grid=(_B, _NC),
        in_specs=in_specs,
        out_specs=out_specs,
        out_shape=out_types,
        compiler_params=pltpu.CompilerParams(
            dimension_semantics=("arbitrary", "arbitrary")),
    )(x4t, qm_row, r_row, t_row, m_row,
      vw4, t_big,
      variable_hyperedge_embedding, patch_hyperedge_embedding)

    v_i32 = variable_indices_flattened
    m_f32 = m_i.astype(f32)
    row_ids = (jnp.arange(_ENC_IN, dtype=jnp.int32)
               .reshape(_ENC_IN // 8, 8)[:, :, None]
               .repeat(_SCL, axis=2)
               .reshape(_ENC_IN // 8, 8 * _SCL))

    @pl.kernel(
        out_type=jax.ShapeDtypeStruct((_B, _ENC_IN, _N), f32),
        mesh=plsc.VectorSubcoreMesh(core_axis_name="c", subcore_axis_name="s"),
        scratch_types=[])
    def _vinc_sc(v_hbm, m_hbm, rid_hbm, o_hbm):
        def body(v_vmem, m_vmem, rid_vmem, o_vmem):
            @pl.loop(0, _SCN, step=_SCL)
            def _(j):
                iv = v_vmem[0, pl.ds(j, _SCL)]
                mv = m_vmem[0, pl.ds(j, _SCL)]
                for rr in range(8):
                    rv = rid_vmem[0, pl.ds(rr * _SCL, _SCL)]
                    o_vmem[0, rr, pl.ds(j, _SCL)] = jnp.where(
                        iv == rv, mv, 0.0)

        pltpu.emit_pipeline(
            body,
            grid=(_B, _ENC_IN // 8, _N // _SCN),
            in_specs=[
                pl.BlockSpec((1, _SCN), lambda b, v, h: (b, h)),
                pl.BlockSpec((1, _SCN), lambda b, v, h: (b, h)),
                pl.BlockSpec((1, 8 * _SCL), lambda b, v, h: (v, 0)),
            ],
            out_specs=[
                pl.BlockSpec((1, 8, _SCN), lambda b, v, h: (b, v, h)),
            ],
            core_axis_name=("c", "s"),
            dimension_semantics=(pltpu.PARALLEL, pltpu.PARALLEL,
                                 pltpu.PARALLEL),
        )(v_hbm, m_hbm, rid_hbm, o_hbm)

    vinc = _vinc_sc(v_i32, m_f32, row_ids)

    return obs, ph, vh, pinc, vinc


# speedup vs baseline: 2.6772x; 2.6772x over previous
import jax
import jax.numpy as jnp
from jax.experimental import pallas as pl
from jax.experimental.pallas import tpu as pltpu
from jax.experimental.pallas import tpu_sc as plsc

_B = 16
_N = 4096
_ENC_IN = 128
_D = 128
_HALF = _D // 2
_PATCH_LEN = 128
_NP = 32
_Q = 64
_NB = 4096
_NC = _N // _NB
_SCL = 16
_SCN = 2048

_TRANS_LHS = (((0,), (0,)), ((), ()))


_CH = 512


def _fused_body(x4t_ref, qm_row_ref, r_row_ref, t_row_ref, v_row_ref,
                m_row_ref, vw_ref, tt_ref,
                obs_ref, pinc_ref, vinc_ref):
    f32 = jnp.float32
    bf16 = jnp.bfloat16
    sub = jax.lax.broadcasted_iota(jnp.int32, (8, _CH), 0)

    for c in range(_NB // _CH):
        sl = pl.ds(c * _CH, _CH)

        val = jax.lax.dot_general(x4t_ref[0, :, sl], vw_ref[...], _TRANS_LHS,
                                  preferred_element_type=f32)
        obs_ref[0, sl, 0:_HALF] = val

        d8q = jnp.broadcast_to(qm_row_ref[0, :, sl], (8, _CH)) - sub
        d8r = jnp.broadcast_to(r_row_ref[0, :, sl], (8, _CH)) - sub
        parts = [(d8q == 8 * k).astype(f32) for k in range(_Q // 8)]
        parts += [(d8r == 8 * k).astype(f32) for k in range(_Q // 8)]
        oht = jnp.concatenate(parts, axis=0).astype(bf16)
        og = jax.lax.dot_general(oht, tt_ref[...], _TRANS_LHS,
                                 preferred_element_type=f32).astype(bf16)
        p = og[:, 0:_D] * og[:, _D:2 * _D]
        time_feat = p[:, 0:_HALF] + p[:, _HALF:_D]
        obs_ref[0, sl, _HALF:_D] = time_feat.astype(f32)

        m8 = jnp.broadcast_to(m_row_ref[0, :, sl].astype(f32), (8, _CH))
        d8v = jnp.broadcast_to(v_row_ref[0, :, sl], (8, _CH)) - sub
        d8p = jnp.broadcast_to(t_row_ref[0, :, sl] // _PATCH_LEN,
                               (8, _CH)) - sub
        for k in range(_ENC_IN // 8):
            vinc_ref[0, 8 * k:8 * (k + 1), sl] = jnp.where(
                d8v == 8 * k, m8, 0.0)
        for k in range(_NP // 8):
            pinc_ref[0, 8 * k:8 * (k + 1), sl] = jnp.where(
                d8p == 8 * k, m8, 0.0)


def kernel(x_flattened, time_indices_flattened, variable_indices_flattened,
           observation_mask_flattened, W_val, b_val, W_time, b_time,
           variable_hyperedge_embedding, patch_hyperedge_embedding):
    f32 = jnp.float32
    bf16 = jnp.bfloat16
    t_i = time_indices_flattened
    m_i = observation_mask_flattened

    qm_row = jnp.where(m_i != 0, t_i >> 6, -1).reshape(_B, 1, _N)
    r_row = (t_i & (_Q - 1)).reshape(_B, 1, _N)
    xm = x_flattened * m_i.astype(f32)
    xh = xm.astype(bf16)
    xl = (xm - xh.astype(f32)).astype(bf16)
    x4t = jnp.stack([xh, xh, xl, m_i.astype(bf16)], axis=1)

    t_row = t_i.reshape(_B, 1, _N)
    v_row = variable_indices_flattened.reshape(_B, 1, _N)
    m_row = m_i.reshape(_B, 1, _N)

    wh = W_val.astype(bf16)
    wl = (W_val - wh.astype(f32)).astype(bf16)
    vw4 = jnp.concatenate([wh, wl, wh, b_val.astype(bf16)[None]], axis=0)

    w_t = W_time[0]
    steps = jnp.arange(_Q, dtype=f32)[:, None]
    a_tab = (_Q * steps) * w_t[None, :]
    b_tab = steps * w_t[None, :] + b_time[None, :]
    qt = jnp.concatenate([jnp.sin(a_tab), jnp.cos(a_tab)], axis=1)
    rt = jnp.concatenate([jnp.cos(b_tab), jnp.sin(b_tab)], axis=1)
    zz = jnp.zeros((_Q, _D), f32)
    t_big = jnp.block([[qt, zz], [zz, rt]]).astype(bf16)

    row_spec = pl.BlockSpec((1, 1, _NB), lambda b, c: (b, 0, c))
    small = lambda shape: pl.BlockSpec(shape, lambda b, c: (0,) * len(shape))

    out_types = (
        jax.ShapeDtypeStruct((_B, _N, _D), f32),
        jax.ShapeDtypeStruct((_B, _NP, _N), f32),
        jax.ShapeDtypeStruct((_B, _ENC_IN, _N), f32),
    )
    out_specs = (
        pl.BlockSpec((1, _NB, _D), lambda b, c: (b, c, 0)),
        pl.BlockSpec((1, _NP, _NB), lambda b, c: (b, 0, c)),
        pl.BlockSpec((1, _ENC_IN, _NB), lambda b, c: (b, 0, c)),
    )
    in_specs = [
        pl.BlockSpec((1, 4, _NB), lambda b, c: (b, 0, c)),
        row_spec, row_spec, row_spec, row_spec, row_spec,
        small((4, _HALF)), small((_D, 2 * _D)),
    ]

    obs, pinc, vinc = pl.pallas_call(
        _fused_body,
        grid=(_B, _NC),
        in_specs=in_specs,
        out_specs=out_specs,
        out_shape=out_types,
        compiler_params=pltpu.CompilerParams(
            dimension_semantics=("arbitrary", "arbitrary")),
    )(x4t, qm_row, r_row, t_row, v_row, m_row,
      vw4, t_big)

    @pl.kernel(
        out_type=(jax.ShapeDtypeStruct((_B, _ENC_IN, _D), f32),
                  jax.ShapeDtypeStruct((_B, _NP, _D), f32)),
        mesh=plsc.ScalarSubcoreMesh(axis_name="c", num_cores=2),
        scratch_types=[pltpu.SemaphoreType.DMA])
    def _hyper_sc(vtab_hbm, ptab_hbm, vh_hbm, ph_hbm, sem):
        core = jax.lax.axis_index("c")
        handles = []
        for i in range(_B // 2):
            b = core * (_B // 2) + i
            handles.append(pltpu.async_copy(vtab_hbm, vh_hbm.at[b], sem))
            handles.append(pltpu.async_copy(ptab_hbm, ph_hbm.at[b], sem))
        for h in handles:
            h.wait()

    vh, ph = _hyper_sc(variable_hyperedge_embedding,
                       patch_hyperedge_embedding)

    return obs, ph, vh, pinc, vinc


# speedup vs baseline: 3.5687x; 1.3330x over previous
import jax
import jax.numpy as jnp
from jax.experimental import pallas as pl
from jax.experimental.pallas import tpu as pltpu

_B = 16
_N = 4096
_ENC_IN = 128
_D = 128
_HALF = _D // 2
_PATCH_LEN = 128
_NP = 32
_Q = 64
_NB = 4096
_NC = _N // _NB

_TRANS_LHS = (((0,), (0,)), ((), ()))


_CH = 1024


def _fused_body(x4t_ref, qm_row_ref, r_row_ref, t_row_ref, v_row_ref,
                m_row_ref, vw_ref, tt_ref, vtab_ref, ptab_ref,
                obs_ref, ph_ref, vh_ref, pinc_ref, vinc_ref):
    f32 = jnp.float32
    bf16 = jnp.bfloat16
    sub = jax.lax.broadcasted_iota(jnp.int32, (8, _CH), 0)

    for c in range(_NB // _CH):
        sl = pl.ds(c * _CH, _CH)

        val = jax.lax.dot_general(x4t_ref[0, :, sl], vw_ref[...], _TRANS_LHS,
                                  preferred_element_type=f32)
        obs_ref[0, sl, 0:_HALF] = val

        d8q = jnp.broadcast_to(qm_row_ref[0, :, sl], (8, _CH)) - sub
        d8r = jnp.broadcast_to(r_row_ref[0, :, sl], (8, _CH)) - sub
        parts = [(d8q == 8 * k).astype(f32) for k in range(_Q // 8)]
        parts += [(d8r == 8 * k).astype(f32) for k in range(_Q // 8)]
        oht = jnp.concatenate(parts, axis=0).astype(bf16)
        og = jax.lax.dot_general(oht, tt_ref[...], _TRANS_LHS,
                                 preferred_element_type=f32).astype(bf16)
        p = og[:, 0:_D] * og[:, _D:2 * _D]
        time_feat = p[:, 0:_HALF] + p[:, _HALF:_D]
        obs_ref[0, sl, _HALF:_D] = time_feat.astype(f32)

        m8 = jnp.broadcast_to(m_row_ref[0, :, sl].astype(f32), (8, _CH))
        d8v = jnp.broadcast_to(v_row_ref[0, :, sl], (8, _CH)) - sub
        d8p = jnp.broadcast_to(t_row_ref[0, :, sl] // _PATCH_LEN,
                               (8, _CH)) - sub
        for k in range(_ENC_IN // 8):
            vinc_ref[0, 8 * k:8 * (k + 1), sl] = jnp.where(
                d8v == 8 * k, m8, 0.0)
        for k in range(_NP // 8):
            pinc_ref[0, 8 * k:8 * (k + 1), sl] = jnp.where(
                d8p == 8 * k, m8, 0.0)

    vh_ref[0] = vtab_ref[...]
    ph_ref[0] = ptab_ref[...]


def kernel(x_flattened, time_indices_flattened, variable_indices_flattened,
           observation_mask_flattened, W_val, b_val, W_time, b_time,
           variable_hyperedge_embedding, patch_hyperedge_embedding):
    f32 = jnp.float32
    bf16 = jnp.bfloat16
    t_i = time_indices_flattened
    m_i = observation_mask_flattened

    qm_row = jnp.where(m_i != 0, t_i >> 6, -1).reshape(_B, 1, _N)
    r_row = (t_i & (_Q - 1)).reshape(_B, 1, _N)
    xm = x_flattened * m_i.astype(f32)
    xh = xm.astype(bf16)
    xl = (xm - xh.astype(f32)).astype(bf16)
    x4t = jnp.stack([xh, xh, xl, m_i.astype(bf16)], axis=1)

    t_row = t_i.reshape(_B, 1, _N)
    v_row = variable_indices_flattened.reshape(_B, 1, _N)
    m_row = m_i.reshape(_B, 1, _N)

    wh = W_val.astype(bf16)
    wl = (W_val - wh.astype(f32)).astype(bf16)
    vw4 = jnp.concatenate([wh, wl, wh, b_val.astype(bf16)[None]], axis=0)

    w_t = W_time[0]
    steps = jnp.arange(_Q, dtype=f32)[:, None]
    a_tab = (_Q * steps) * w_t[None, :]
    b_tab = steps * w_t[None, :] + b_time[None, :]
    qt = jnp.concatenate([jnp.sin(a_tab), jnp.cos(a_tab)], axis=1)
    rt = jnp.concatenate([jnp.cos(b_tab), jnp.sin(b_tab)], axis=1)
    zz = jnp.zeros((_Q, _D), f32)
    t_big = jnp.block([[qt, zz], [zz, rt]]).astype(bf16)

    row_spec = pl.BlockSpec((1, 1, _NB), lambda b, c: (b, 0, c))
    small = lambda shape: pl.BlockSpec(shape, lambda b, c: (0,) * len(shape))

    out_types = (
        jax.ShapeDtypeStruct((_B, _N, _D), f32),
        jax.ShapeDtypeStruct((_B, _NP, _D), f32),
        jax.ShapeDtypeStruct((_B, _ENC_IN, _D), f32),
        jax.ShapeDtypeStruct((_B, _NP, _N), f32),
        jax.ShapeDtypeStruct((_B, _ENC_IN, _N), f32),
    )
    out_specs = (
        pl.BlockSpec((1, _NB, _D), lambda b, c: (b, c, 0)),
        pl.BlockSpec((1, _NP, _D), lambda b, c: (b, 0, 0)),
        pl.BlockSpec((1, _ENC_IN, _D), lambda b, c: (b, 0, 0)),
        pl.BlockSpec((1, _NP, _NB), lambda b, c: (b, 0, c)),
        pl.BlockSpec((1, _ENC_IN, _NB), lambda b, c: (b, 0, c)),
    )
    in_specs = [
        pl.BlockSpec((1, 4, _NB), lambda b, c: (b, 0, c)),
        row_spec, row_spec, row_spec, row_spec, row_spec,
        small((4, _HALF)), small((_D, 2 * _D)),
        small((_ENC_IN, _D)), small((_NP, _D)),
    ]

    return pl.pallas_call(
        _fused_body,
        grid=(_B, _NC),
        in_specs=in_specs,
        out_specs=out_specs,
        out_shape=out_types,
        compiler_params=pltpu.CompilerParams(
            dimension_semantics=("arbitrary", "arbitrary")),
    )(x4t, qm_row, r_row, t_row, v_row, m_row,
      vw4, t_big,
      variable_hyperedge_embedding, patch_hyperedge_embedding)
